# baseline (speedup 1.0000x reference)
import functools

import jax
import jax.numpy as jnp
import numpy as np
from jax import lax
from jax.experimental import pallas as pl
from jax.experimental.pallas import tpu as pltpu
from jax.experimental.pallas import tpu_sc as plsc

_N = 1_000_000
_D = 16
_NZ = 500_000
_NF = _N * _D
_NC = 2
_NS = 16
_NW = _NC * _NS
_HALF = _NF // _NW
_C = 5000
_GROUPS = _HALF // _C
_RING = 2


@functools.cache
def _build_index_table():
    pos = np.sort(np.asarray(jax.random.permutation(jax.random.key(42), _N)[:_NZ]))
    rows = []
    counts = []
    for h in range(2):
        loc = pos[(pos >= h * _HALF) & (pos < (h + 1) * _HALF)] - h * _HALF
        for j in range(_GROUPS):
            ent = loc[(loc >= j * _C) & (loc < (j + 1) * _C)] - j * _C
            assert len(ent) > 0
            rows.append(ent.astype(np.int32))
            counts.append(len(ent))
    k = int(-(-max(counts) // 16) * 16)
    tab = np.empty((2 * _GROUPS, k), np.int32)
    for i, ent in enumerate(rows):
        tab[i, : len(ent)] = ent
        tab[i, len(ent):] = ent[0]
    return tab


try:
    _K_CH = _build_index_table().shape[1]
except Exception:
    _K_CH = None


@functools.cache
def _get_sc_kernel():
    k_ch = _build_index_table().shape[1]
    mesh = plsc.VectorSubcoreMesh(
        core_axis_name="c", subcore_axis_name="s", num_cores=_NC, num_subcores=_NS
    )

    @functools.partial(
        pl.kernel,
        out_type=(
            jax.ShapeDtypeStruct((_NF,), jnp.float32),
            jax.ShapeDtypeStruct((_NF,), jnp.float32),
        ),
        mesh=mesh,
        compiler_params=pltpu.CompilerParams(
            use_tc_tiling_on_sc=False, needs_layout_passes=False
        ),
        scratch_types=(
            [pltpu.VMEM((_C,), jnp.float32) for _ in range(_RING)]
            + [pltpu.VMEM((k_ch,), jnp.int32) for _ in range(_RING)]
            + [pltpu.SemaphoreType.DMA for _ in range(2 * _RING)]
        ),
    )
    def _sc_dropout(emb1, emb2, idx_hbm, out1, out2, *scratch):
        bufs = scratch[:_RING]
        idxs = scratch[_RING:2 * _RING]
        insems = scratch[2 * _RING:3 * _RING]
        outsems = scratch[3 * _RING:4 * _RING]

        c = lax.axis_index("c")
        s = lax.axis_index("s")
        wid = s * _NC + c
        zvec = jnp.zeros((16,), jnp.float32)

        def fire_in(src, g, b):
            base = wid * _HALF + g * _C
            pltpu.async_copy(src.at[pl.ds(base, _C)], bufs[b], insems[b])
            row = (wid % 2) * _GROUPS + g
            pltpu.async_copy(idx_hbm.at[row], idxs[b], insems[b])

        def process(src, dst, g, b):
            base = wid * _HALF + g * _C
            row = (wid % 2) * _GROUPS + g
            pltpu.make_async_copy(src.at[pl.ds(base, _C)], bufs[b], insems[b]).wait()
            pltpu.make_async_copy(idx_hbm.at[row], idxs[b], insems[b]).wait()

            @pl.loop(0, k_ch // 16)
            def _zero(grp):
                rvec = idxs[b][pl.ds(grp * 16, 16)]
                plsc.store_scatter(bufs[b], [rvec], zvec)

            pltpu.async_copy(bufs[b], dst.at[pl.ds(base, _C)], outsems[b])
            pltpu.make_async_copy(bufs[b], dst.at[pl.ds(base, _C)], outsems[b]).wait()

        srcs = (emb1, emb2)
        dsts = (out1, out2)
        for b in range(_RING):
            fire_in(srcs[b], 0, b)

        @pl.loop(0, _GROUPS)
        def _run(jg):
            for b in range(_RING):
                process(srcs[b], dsts[b], jg, b)

                @pl.when(jg < _GROUPS - 1)
                def _prefetch():
                    fire_in(srcs[b], jg + 1, b)

    return _sc_dropout


def kernel(emb1, emb2):
    idx_tab = jnp.asarray(_build_index_table())
    f1 = emb1.T.reshape(_NF)
    f2 = emb2.T.reshape(_NF)
    o1, o2 = _get_sc_kernel()(f1, f2, idx_tab)
    return o1.reshape(_D, _N).T, o2.reshape(_D, _N).T


# speedup vs baseline: 3.8280x; 3.8280x over previous
import functools

import jax
import jax.numpy as jnp
import numpy as np
from jax import lax
from jax.experimental import pallas as pl
from jax.experimental.pallas import tpu as pltpu
from jax.experimental.pallas import tpu_sc as plsc

_N = 1_000_000
_D = 16
_NZ = 500_000
_NC = 2
_NS = 16
_NW = _NC * _NS
_CH = 128

_PER_W = -(-_NZ // _NW)
_N_CHUNKS = -(-_PER_W // _CH)


@functools.cache
def _build_index_table() -> np.ndarray:
    idx = np.sort(np.asarray(jax.random.permutation(jax.random.key(42), _N)[:_NZ]))
    k = _N_CHUNKS * _CH
    tab = np.empty((_NW, k), np.int32)
    for w in range(_NW):
        part = idx[w * _PER_W:(w + 1) * _PER_W]
        tab[w, : len(part)] = part
        tab[w, len(part):] = part[-1]
    return tab.reshape(_NW, _N_CHUNKS, _CH)


try:
    _build_index_table()
except Exception:
    pass


@functools.cache
def _get_sc_zero_rows():
    mesh = plsc.VectorSubcoreMesh(
        core_axis_name="c", subcore_axis_name="s", num_cores=_NC, num_subcores=_NS
    )

    @functools.partial(
        pl.kernel,
        mesh=mesh,
        compiler_params=pltpu.CompilerParams(use_tc_tiling_on_sc=False),
        scratch_types=[
            pltpu.VMEM((_N_CHUNKS, _CH), jnp.int32),
            pltpu.VMEM((_CH, _D), jnp.float32),
            pltpu.SemaphoreType.DMA,
            pltpu.SemaphoreType.DMA,
        ],
    )
    def _sc_zero_rows(idx_hbm, zeros_hbm, out1, out2, idx_v, zeros_v, lsem, ssem):
        c = lax.axis_index("c")
        s = lax.axis_index("s")
        wid = s * _NC + c

        pltpu.async_copy(zeros_hbm, zeros_v, lsem).wait()
        pltpu.async_copy(idx_hbm.at[wid], idx_v, lsem).wait()

        @pl.loop(0, _N_CHUNKS)
        def _fire(j):
            pltpu.async_copy(zeros_v, out1.at[idx_v.at[j]], ssem)
            pltpu.async_copy(zeros_v, out2.at[idx_v.at[j]], ssem)

        @pl.loop(0, _N_CHUNKS)
        def _drain(j):
            pltpu.make_async_copy(zeros_v, out1.at[idx_v.at[j]], ssem).wait()
            pltpu.make_async_copy(zeros_v, out2.at[idx_v.at[j]], ssem).wait()

    return _sc_zero_rows


def kernel(emb1, emb2):
    idx_tab = jnp.asarray(_build_index_table())
    zeros = jnp.zeros((_CH, _D), jnp.float32)
    out1 = jax.new_ref(emb1)
    out2 = jax.new_ref(emb2)
    _get_sc_zero_rows()(idx_tab, zeros, out1, out2)
    return out1[...], out2[...]
